# baseline (speedup 1.0000x reference)
import functools

import jax
import jax.numpy as jnp
from jax import lax
from jax.experimental import pallas as pl
from jax.experimental.pallas import tpu as pltpu
from jax.experimental.pallas import tpu_sc as plsc

_B = 4
_N = 2048
_S = 1024
_BS = 128


def _dist_body(xj_ref, yj_ref, zj_ref, xi_ref, yi_ref, zi_ref, o_ref):
    xj = xj_ref[...][:, None, :]
    yj = yj_ref[...][:, None, :]
    zj = zj_ref[...][:, None, :]
    xi = xi_ref[...][:, :, None]
    yi = yi_ref[...][:, :, None]
    zi = zi_ref[...][:, :, None]
    dx = xj - xi
    dy = yj - yi
    dz = zj - zi
    s = dx * dx + dy * dy
    s = s + dz * dz
    o_ref[...] = jnp.sqrt(jnp.maximum(s, 1e-12))


def _tc_dist(xs, ys, zs):
    full = pl.BlockSpec((_B, _N), lambda i: (0, 0))
    blk = pl.BlockSpec((_B, _BS), lambda i: (0, i))
    return pl.pallas_call(
        _dist_body,
        grid=(_N // _BS,),
        in_specs=[full, full, full, blk, blk, blk],
        out_specs=pl.BlockSpec((_B, _BS, _N), lambda i: (0, i, 0)),
        out_shape=jax.ShapeDtypeStruct((_B, _N, _N), jnp.float32),
    )(xs, ys, zs, xs, ys, zs)


def _sc_loop(d16, p2):
    mesh = plsc.VectorSubcoreMesh(core_axis_name="c", subcore_axis_name="s")
    big = jnp.int32(1 << 30)
    ninf = jnp.float32(-jnp.inf)
    dnums = lax.GatherDimensionNumbers(
        offset_dims=(), collapsed_slice_dims=(0,), start_index_map=(0,)
    )

    @functools.partial(
        pl.kernel,
        mesh=mesh,
        out_type=jax.ShapeDtypeStruct((_B, _S, 128), jnp.float32),
        scratch_types=[
            pltpu.VMEM((16, 128), jnp.float32),
            pltpu.VMEM((16, 128), jnp.float32),
            pltpu.VMEM((16,), jnp.int32),
            pltpu.VMEM((_S + 16,), jnp.int32),
            pltpu.VMEM((128, 128), jnp.float32),
            pltpu.SemaphoreType.DMA,
        ],
    )
    def k(d_hbm, p_hbm, out_hbm, ds_v, row_v, idx_v, wids_v, pt_v, sem):
        wid = lax.axis_index("s") * 2 + lax.axis_index("c")

        @pl.when(wid < _B)
        def _():
            base = wid * _N
            lane = lax.iota(jnp.int32, 16)

            def perm(v, idx):
                return lax.gather(
                    v, jnp.reshape(idx, (16, 1)), dnums, (1,),
                    mode=lax.GatherScatterMode.PROMISE_IN_BOUNDS,
                )

            def allmax_f32(v):
                for s in (1, 2, 4, 8):
                    v = jnp.maximum(v, perm(v, lane ^ s))
                return v

            def allmin_i32(v):
                for s in (1, 2, 4, 8):
                    v = jnp.minimum(v, perm(v, lane ^ s))
                return v

            def initf(c, _):
                sl = pl.ds(c * 16, 16)
                inf16 = jnp.full((16,), jnp.inf, jnp.float32)
                for r in range(16):
                    ds_v[r, sl] = inf16
                return 0

            lax.fori_loop(0, 8, initf, 0)
            idx_v[...] = base * 16 + lane
            pltpu.async_copy(d_hbm.at[idx_v], row_v, sem).wait()

            def step(kk, _):
                cm = jnp.full((16,), ninf, jnp.float32)
                cg = jnp.full((16,), big, jnp.int32)
                for r in range(16):
                    def fuse(c, carry, r=r):
                        fm, fg = carry
                        sl = pl.ds(c * 16, 16)
                        v = jnp.minimum(ds_v[r, sl], row_v[r, sl])
                        ds_v[r, sl] = v
                        upd = v > fm
                        gvec = lane + (r * 128) + c * 16
                        return jnp.where(upd, v, fm), jnp.where(upd, gvec, fg)

                    cm, cg = lax.fori_loop(0, 8, fuse, (cm, cg))
                m_all = allmax_f32(cm)
                cand = jnp.where(cm == m_all, cg, big)
                w_all = allmin_i32(cand)
                rowid = base + w_all
                wids_v[pl.ds(kk, 16)] = rowid
                idx_v[...] = rowid * 16 + lane
                pltpu.async_copy(d_hbm.at[idx_v], row_v, sem).wait()
                return 0

            lax.fori_loop(0, _S, step, 0)
            for g in range(8):
                pltpu.async_copy(
                    p_hbm.at[wids_v.at[pl.ds(g * 128, 128)]], pt_v, sem
                ).wait()
                pltpu.sync_copy(pt_v, out_hbm.at[wid, pl.ds(g * 128, 128)])

    return k(d16, p2)


def kernel(inputs):
    xs = inputs[:, :, 0]
    ys = inputs[:, :, 1]
    zs = inputs[:, :, 2]
    d16 = _tc_dist(xs, ys, zs).reshape(_B * _N * 16, 128)
    p2 = jnp.pad(inputs.reshape(_B * _N, 3), ((0, 0), (0, 125)))
    out = _sc_loop(d16, p2)
    return out[:, :, :3]


# speedup vs baseline: 6.1669x; 6.1669x over previous
import jax
import jax.numpy as jnp
from jax.experimental import pallas as pl
from jax.experimental.pallas import tpu as pltpu

_B = 4
_N = 2048
_S = 1024
_H = _N // 2
_R = 2 * _B


def _fps_body(x_ref, y_ref, z_ref, xh_ref, xl_ref, yh_ref, yl_ref, zh_ref, zl_ref):
    X = x_ref[...]
    Y = y_ref[...]
    Z = z_ref[...]
    half = jax.lax.broadcasted_iota(jnp.int32, (_R, _H), 0) >> 2
    gidx = jax.lax.broadcasted_iota(jnp.int32, (_R, _H), 1) + half * _H
    gs = jnp.bitwise_or(jnp.int32(0x20000000), jax.lax.shift_left(gidx, 18))
    c16 = jnp.int32(0xFFFF)
    big = jnp.float32(4.0)

    def pswap(v):
        return pltpu.roll(v, 4, 0)

    def make_keys(V):
        b = jax.lax.bitcast_convert_type(V, jnp.int32)
        hi = jax.lax.shift_right_logical(b, 16)
        lo = jnp.bitwise_and(b, c16)
        return (
            jax.lax.bitcast_convert_type(jnp.bitwise_or(gs, hi), jnp.float32),
            jax.lax.bitcast_convert_type(jnp.bitwise_or(gs, lo), jnp.float32),
        )

    kxh, kxl = make_keys(X)
    kyh, kyl = make_keys(Y)
    kzh, kzl = make_keys(Z)

    def dist_from(px, py, pz):
        dx = X - px
        dy = Y - py
        dz = Z - pz
        s = dx * dx + dy * dy
        s = s + dz * dz
        return jnp.sqrt(jnp.maximum(s, 1e-12))

    half1 = half[:, 0:1]

    def pair0(col):
        return jnp.where(half1 == 0, col, pswap(col))

    ds0 = dist_from(
        pair0(x_ref[:, 0:1]), pair0(y_ref[:, 0:1]), pair0(z_ref[:, 0:1])
    )

    def body(k, ds):
        dsm = jnp.maximum(ds, pswap(ds))
        m = jnp.max(dsm, axis=1, keepdims=True)
        elig = ds == m

        def ext(kh, kl, oh_ref, ol_ref):
            keh = jnp.where(elig, kh, big)
            kel = jnp.where(elig, kl, big)
            keh = jnp.minimum(keh, pswap(keh))
            kel = jnp.minimum(kel, pswap(kel))
            rh = jnp.min(keh, axis=1, keepdims=True)
            rl = jnp.min(kel, axis=1, keepdims=True)
            oh_ref[pl.ds(k, 1)] = rh[None]
            ol_ref[pl.ds(k, 1)] = rl[None]
            rhb = jax.lax.bitcast_convert_type(rh, jnp.int32)
            rlb = jax.lax.bitcast_convert_type(rl, jnp.int32)
            bits = jnp.bitwise_or(
                jax.lax.shift_left(jnp.bitwise_and(rhb, c16), 16),
                jnp.bitwise_and(rlb, c16),
            )
            return jax.lax.bitcast_convert_type(bits, jnp.float32)

        px = ext(kxh, kxl, xh_ref, xl_ref)
        py = ext(kyh, kyl, yh_ref, yl_ref)
        pz = ext(kzh, kzl, zh_ref, zl_ref)
        return jnp.minimum(ds, dist_from(px, py, pz))

    jax.lax.fori_loop(0, _S, body, ds0)


def kernel(inputs):
    def fold(v):
        return v.reshape(_B, 2, _H).swapaxes(0, 1).reshape(_R, _H)

    xs = fold(inputs[:, :, 0])
    ys = fold(inputs[:, :, 1])
    zs = fold(inputs[:, :, 2])
    shape = jax.ShapeDtypeStruct((_S, _R, 1), jnp.float32)
    outs = pl.pallas_call(
        _fps_body,
        out_shape=(shape,) * 6,
    )(xs, ys, zs)

    def unpack(rh8, rl8):
        rh = jnp.minimum(rh8[:, :_B, 0], rh8[:, _B:, 0])
        rl = jnp.minimum(rl8[:, :_B, 0], rl8[:, _B:, 0])
        rhb = jax.lax.bitcast_convert_type(rh, jnp.int32)
        rlb = jax.lax.bitcast_convert_type(rl, jnp.int32)
        bits = jnp.bitwise_or(
            jax.lax.shift_left(jnp.bitwise_and(rhb, 0xFFFF), 16),
            jnp.bitwise_and(rlb, 0xFFFF),
        )
        return jax.lax.bitcast_convert_type(bits, jnp.float32)

    px = unpack(outs[0], outs[1])
    py = unpack(outs[2], outs[3])
    pz = unpack(outs[4], outs[5])
    out = jnp.stack([px, py, pz], axis=-1)
    return jnp.transpose(out, (1, 0, 2))


# speedup vs baseline: 6.2349x; 1.0110x over previous
import jax
import jax.numpy as jnp
from jax.experimental import pallas as pl

_B = 4
_N = 2048
_S = 1024


def _fps_body(x_ref, y_ref, z_ref, ox_ref, oy_ref, oz_ref):
    X = x_ref[...]
    Y = y_ref[...]
    Z = z_ref[...]
    gidx = jax.lax.broadcasted_iota(jnp.int32, (_B, _N), 1)
    gs = jnp.bitwise_or(jnp.int32(0x20000000), jax.lax.shift_left(gidx, 18))
    c16 = jnp.int32(0xFFFF)
    big = jnp.float32(4.0)

    def make_keys(V):
        b = jax.lax.bitcast_convert_type(V, jnp.int32)
        hi = jax.lax.shift_right_logical(b, 16)
        lo = jnp.bitwise_and(b, c16)
        return (
            jax.lax.bitcast_convert_type(jnp.bitwise_or(gs, hi), jnp.float32),
            jax.lax.bitcast_convert_type(jnp.bitwise_or(gs, lo), jnp.float32),
        )

    kxh, kxl = make_keys(X)
    kyh, kyl = make_keys(Y)
    kzh, kzl = make_keys(Z)

    def dist_from(px, py, pz):
        dx = X - px
        dy = Y - py
        dz = Z - pz
        s = dx * dx + dy * dy
        s = s + dz * dz
        return jnp.sqrt(jnp.maximum(s, 1e-12))

    ds0 = dist_from(x_ref[:, 0:1], y_ref[:, 0:1], z_ref[:, 0:1])

    def body(k, ds):
        m = jnp.max(ds, axis=1, keepdims=True)
        elig = ds == m

        def ext(kh, kl):
            rh = jnp.min(jnp.where(elig, kh, big), axis=1, keepdims=True)
            rl = jnp.min(jnp.where(elig, kl, big), axis=1, keepdims=True)
            rhb = jax.lax.bitcast_convert_type(rh, jnp.int32)
            rlb = jax.lax.bitcast_convert_type(rl, jnp.int32)
            bits = jnp.bitwise_or(
                jax.lax.shift_left(jnp.bitwise_and(rhb, c16), 16),
                jnp.bitwise_and(rlb, c16),
            )
            return jax.lax.bitcast_convert_type(bits, jnp.float32)

        px = ext(kxh, kxl)
        py = ext(kyh, kyl)
        pz = ext(kzh, kzl)
        ox_ref[pl.ds(k, 1)] = px[None]
        oy_ref[pl.ds(k, 1)] = py[None]
        oz_ref[pl.ds(k, 1)] = pz[None]
        return jnp.minimum(ds, dist_from(px, py, pz))

    jax.lax.fori_loop(0, _S, body, ds0)


def kernel(inputs):
    xs = inputs[:, :, 0]
    ys = inputs[:, :, 1]
    zs = inputs[:, :, 2]
    shape = jax.ShapeDtypeStruct((_S, _B, 1), jnp.float32)
    ox, oy, oz = pl.pallas_call(
        _fps_body,
        out_shape=(shape, shape, shape),
    )(xs, ys, zs)
    out = jnp.concatenate([ox, oy, oz], axis=-1)
    return jnp.transpose(out, (1, 0, 2))
